# baseline (speedup 1.0000x reference)
import jax
import jax.numpy as jnp
from jax import lax
from jax.experimental import pallas as pl
from jax.experimental.pallas import tpu as pltpu
from jax.experimental.pallas import tpu_sc as plsc

_BATCH = 4096
_SEQ = 200
_EMBED = 64
_LANES = 16
_NC = 2
_NS = 16
_NW = _NC * _NS
_ROWS_PER_W = _BATCH // _NW
_HALF = _SEQ // 2


def _sc_body(x_hbm, tok_hbm, pos_hbm, out_hbm, idx_v, rows_v, pos_v, sem):
    wid = lax.axis_index("s") * _NC + lax.axis_index("c")
    base = wid * _ROWS_PER_W
    pltpu.sync_copy(pos_hbm, pos_v)

    def row_body(r, carry):
        row = base + r
        pltpu.sync_copy(x_hbm.at[row], idx_v)
        pltpu.sync_copy(pos_hbm, rows_v)
        cp0 = pltpu.async_copy(
            tok_hbm.at[idx_v.at[0]], rows_v.at[pl.ds(0, _HALF)], sem,
            add=True)
        cp1 = pltpu.async_copy(
            tok_hbm.at[idx_v.at[1]], rows_v.at[pl.ds(_HALF, _HALF)], sem,
            add=True)
        cp0.wait()
        cp1.wait()
        pltpu.sync_copy(rows_v, out_hbm.at[row])
        return carry

    lax.fori_loop(0, _ROWS_PER_W, row_body, 0)


def kernel(x, token_table, pos_table):
    x32 = x.astype(jnp.int32).reshape(_BATCH, 2, _HALF)
    f = pl.kernel(
        _sc_body,
        mesh=plsc.VectorSubcoreMesh(core_axis_name="c", subcore_axis_name="s"),
        compiler_params=pltpu.CompilerParams(use_tc_tiling_on_sc=False),
        out_type=jax.ShapeDtypeStruct((_BATCH, _SEQ, _EMBED), jnp.float32),
        scratch_types=[
            pltpu.VMEM((2, _HALF), jnp.int32),
            pltpu.VMEM((_SEQ, _EMBED), jnp.float32),
            pltpu.VMEM((_SEQ, _EMBED), jnp.float32),
            pltpu.SemaphoreType.DMA,
        ],
    )
    return f(x32, token_table, pos_table)


# speedup vs baseline: 1.6764x; 1.6764x over previous
import jax
import jax.numpy as jnp
from jax import lax
from jax.experimental import pallas as pl
from jax.experimental.pallas import tpu as pltpu
from jax.experimental.pallas import tpu_sc as plsc

_BATCH = 4096
_SEQ = 200
_EMBED = 64
_LANES = 16
_NC = 2
_NS = 16
_NW = _NC * _NS
_ROWS_PER_W = _BATCH // _NW
_HALF = _SEQ // 2
_NBUF = 4


def _sc_body(x_hbm, tok_hbm, pos_hbm, out_hbm, idx_v, rows_v, pos_v, *sems):
    semg = sems[:_NBUF]
    semo = sems[_NBUF:]
    wid = lax.axis_index("s") * _NC + lax.axis_index("c")
    base = wid * _ROWS_PER_W
    pltpu.sync_copy(pos_hbm, pos_v)
    pltpu.sync_copy(x_hbm.at[pl.ds(base, _ROWS_PER_W)], idx_v)

    def fire_gather(s, b):
        pltpu.async_copy(
            tok_hbm.at[idx_v.at[s, 0]], rows_v.at[b, pl.ds(0, _HALF)],
            semg[b])
        pltpu.async_copy(
            tok_hbm.at[idx_v.at[s, 1]], rows_v.at[b, pl.ds(_HALF, _HALF)],
            semg[b])

    _LEAD = 2
    for b in range(_LEAD):
        fire_gather(b, b)

    def pair_body(it, carry):
        for b in range(_NBUF):
            s = it * _NBUF + b
            s_f = s + _LEAD
            bf = (b + _LEAD) % _NBUF

            def fire_next():
                pltpu.make_async_copy(
                    rows_v.at[bf], out_hbm.at[base], semo[bf]).wait()
                fire_gather(s_f, bf)

            def fire_first():
                fire_gather(s_f, bf)

            if b < _LEAD:
                lax.cond(it > 0, fire_next, fire_first)
            else:
                pl.when(it < (_ROWS_PER_W // _NBUF) - 1)(fire_next)

            pltpu.make_async_copy(pos_hbm, rows_v.at[b], semg[b]).wait()

            def add_body(i, c2):
                for d in range(_EMBED // _LANES):
                    sl = pl.ds(d * _LANES, _LANES)
                    rows_v[b, i, sl] = rows_v[b, i, sl] + pos_v[i, sl]
                return c2

            lax.fori_loop(0, _SEQ, add_body, 0)
            pltpu.async_copy(rows_v.at[b], out_hbm.at[base + s], semo[b])
        return carry

    lax.fori_loop(0, _ROWS_PER_W // _NBUF, pair_body, 0)
    for b in range(_NBUF):
        pltpu.make_async_copy(rows_v.at[b], out_hbm.at[base], semo[b]).wait()


def kernel(x, token_table, pos_table):
    x32 = x.astype(jnp.int32).reshape(_BATCH, 2, _HALF)
    f = pl.kernel(
        _sc_body,
        mesh=plsc.VectorSubcoreMesh(core_axis_name="c", subcore_axis_name="s"),
        compiler_params=pltpu.CompilerParams(use_tc_tiling_on_sc=False),
        out_type=jax.ShapeDtypeStruct((_BATCH, _SEQ, _EMBED), jnp.float32),
        scratch_types=[
            pltpu.VMEM((_ROWS_PER_W, 2, _HALF), jnp.int32),
            pltpu.VMEM((_NBUF, _SEQ, _EMBED), jnp.float32),
            pltpu.VMEM((_SEQ, _EMBED), jnp.float32),
        ] + [pltpu.SemaphoreType.DMA] * (2 * _NBUF),
    )
    return f(x32, token_table, pos_table)
